# baseline (speedup 1.0000x reference)
import functools

import jax
import jax.numpy as jnp
from jax import lax
from jax.experimental import pallas as pl
from jax.experimental.pallas import tpu as pltpu
from jax.experimental.pallas import tpu_sc as plsc

N = 10000
E = 320000
D = 128

NC = 2
NS = 16
NW = NC * NS
L = 16

EPW = E // NW
K = 80
NCHUNK = EPW // K
SPS = 624
TAIL = N - SPS * NS
ZROWS = 16
NZ = SPS // ZROWS
KA = 128
NCA = 79
EPAD = NW * NCA * KA


def _sc_mesh():
    return plsc.VectorSubcoreMesh(core_axis_name="c", subcore_axis_name="s")


@functools.partial(
    pl.kernel,
    mesh=_sc_mesh(),
    out_type=jax.ShapeDtypeStruct((NC, N, L), jnp.float32),
    scratch_types=[
        pltpu.VMEM((1, K), jnp.int32),
        pltpu.VMEM((K,), jnp.float32),
        pltpu.VMEM((K, L), jnp.float32),
        pltpu.VMEM((ZROWS, L), jnp.float32),
        pltpu.VMEM_SHARED((N, L), jnp.float32),
    ],
)
def _deg_kernel(dst_hbm, ewp_hbm, out_hbm, dstv, pbuf, msg, zbuf, deg_sp):
    c = lax.axis_index("c")
    s = lax.axis_index("s")
    w = s * NC + c
    zero = jnp.zeros((L,), jnp.float32)
    for r in range(ZROWS):
        zbuf[r, :] = zero

    def zloop(i, carry):
        pltpu.sync_copy(zbuf, deg_sp.at[pl.ds(s * SPS + i * ZROWS, ZROWS)])
        return carry

    lax.fori_loop(0, NZ, zloop, 0)

    @pl.when(s == NS - 1)
    def _():
        pltpu.sync_copy(zbuf, deg_sp.at[pl.ds(SPS * NS, TAIL)])

    plsc.subcore_barrier()

    def body(cidx, carry):
        base = w * EPW + cidx * K
        pltpu.sync_copy(dst_hbm.at[pl.ds(base, K)], dstv.at[0])
        pltpu.sync_copy(ewp_hbm.at[pl.ds(base, K)], pbuf)
        ramp = lax.iota(jnp.int32, L)
        onehot = [jnp.where(ramp == i, 1.0, 0.0).astype(jnp.float32)
                  for i in range(L)]

        def fill(g, carry2):
            wvec = jnp.exp(pbuf[pl.ds(g * L, L)])
            for i in range(L):
                msg[g * L + i, pl.ds(0, L)] = wvec * onehot[i]
            return carry2

        lax.fori_loop(0, K // L, fill, 0)
        pltpu.sync_copy(msg, deg_sp.at[dstv.at[0]], add=True)
        return carry

    lax.fori_loop(0, NCHUNK, body, 0)
    plsc.subcore_barrier()
    pltpu.sync_copy(deg_sp.at[pl.ds(s * SPS, SPS)],
                    out_hbm.at[c, pl.ds(s * SPS, SPS)])

    @pl.when(s == NS - 1)
    def _():
        pltpu.sync_copy(deg_sp.at[pl.ds(SPS * NS, TAIL)],
                        out_hbm.at[c, pl.ds(SPS * NS, TAIL)])


@functools.partial(
    pl.kernel,
    mesh=_sc_mesh(),
    out_type=jax.ShapeDtypeStruct((NC, N, D), jnp.float32),
    scratch_types=[
        pltpu.VMEM((2, KA), jnp.int32),
        pltpu.VMEM((2, KA), jnp.int32),
        pltpu.VMEM((KA,), jnp.float32),
        pltpu.VMEM((KA,), jnp.float32),
        pltpu.VMEM((KA, D), jnp.float32),
        pltpu.VMEM((KA, D), jnp.float32),
        pltpu.VMEM((ZROWS, D), jnp.float32),
        pltpu.VMEM_SHARED((N, D), jnp.float32),
        pltpu.SemaphoreType.DMA((2,)),
        pltpu.SemaphoreType.DMA((2,)),
    ],
)
def _agg_kernel(hp_hbm, idx_hbm, ewp_hbm, out_hbm,
                idxb0, idxb1, pbuf0, pbuf1, rows0, rows1, zbuf, agg_sp,
                gsems, isems):
    c = lax.axis_index("c")
    s = lax.axis_index("s")
    w = s * NC + c

    pltpu.sync_copy(idx_hbm.at[w, 0], idxb0)
    pltpu.sync_copy(ewp_hbm.at[w, 0], pbuf0)
    pltpu.async_copy(hp_hbm.at[idxb0.at[0]], rows0, gsems.at[0])

    zero = jnp.zeros((L,), jnp.float32)
    for r in range(ZROWS):
        for j in range(D // L):
            zbuf[r, pl.ds(j * L, L)] = zero

    nz = NZ + jnp.where(s == NS - 1, 1, 0)

    def zloop(i, carry):
        pltpu.sync_copy(zbuf, agg_sp.at[pl.ds(s * SPS + i * ZROWS, ZROWS)])
        return carry

    lax.fori_loop(0, nz, zloop, 0)
    plsc.subcore_barrier()

    def scale(buf, pbuf):
        def sgroup(g, carry2):
            wvec = jnp.exp(pbuf[pl.ds(g * L, L)])
            for i in range(L):
                wv = jnp.zeros((L,), jnp.float32) + wvec[i]
                e = g * L + i
                for j in range(D // L):
                    buf[e, pl.ds(j * L, L)] = buf[e, pl.ds(j * L, L)] * wv
            return carry2

        lax.fori_loop(0, KA // L, sgroup, 0)

    def halfstep(t, idxb, pbuf, rows, idxn, pbufn, rowsn, bsel, bnext):
        @pl.when(t <= NCA - 2)
        def _():
            pltpu.async_copy(idx_hbm.at[w, t + 1], idxn, isems.at[bnext])
            pltpu.async_copy(ewp_hbm.at[w, t + 1], pbufn, isems.at[bnext])

        pltpu.make_async_copy(hp_hbm.at[idxb.at[0]], rows,
                              gsems.at[bsel]).wait()
        scale(rows, pbuf)
        pltpu.async_copy(rows, agg_sp.at[idxb.at[1]], gsems.at[bsel],
                         add=True).wait()

        @pl.when(t <= NCA - 2)
        def _():
            pltpu.make_async_copy(idx_hbm.at[w, 0], idxn,
                                  isems.at[bnext]).wait()
            pltpu.make_async_copy(ewp_hbm.at[w, 0], pbufn,
                                  isems.at[bnext]).wait()
            pltpu.async_copy(hp_hbm.at[idxn.at[0]], rowsn, gsems.at[bnext])

    def body(t, carry):
        @pl.when(t % 2 == 0)
        def _():
            halfstep(t, idxb0, pbuf0, rows0, idxb1, pbuf1, rows1, 0, 1)

        @pl.when(t % 2 == 1)
        def _():
            halfstep(t, idxb1, pbuf1, rows1, idxb0, pbuf0, rows0, 1, 0)

        return carry

    lax.fori_loop(0, NCA, body, 0)
    plsc.subcore_barrier()

    ncp = NZ + jnp.where(s == NS - 1, 1, 0)

    def cloop(i, carry):
        pltpu.sync_copy(agg_sp.at[pl.ds(s * SPS + i * ZROWS, ZROWS)],
                        out_hbm.at[c, pl.ds(s * SPS + i * ZROWS, ZROWS)])
        return carry

    lax.fori_loop(0, ncp, cloop, 0)


def _tc1_body(x_ref, w1_ref, deg_ref, h1_ref, hp1_ref, dis_ref):
    h1 = lax.dot_general(x_ref[...], w1_ref[...],
                         (((1,), (1,)), ((), ())),
                         preferred_element_type=jnp.float32)
    deg = jnp.sum(deg_ref[0] + deg_ref[1], axis=-1, keepdims=True) + 1.0
    dis = lax.rsqrt(deg)
    h1_ref[...] = h1
    hp1_ref[...] = h1 * dis
    dis_ref[...] = dis


_tc1 = pl.pallas_call(
    _tc1_body,
    out_shape=[
        jax.ShapeDtypeStruct((N, D), jnp.float32),
        jax.ShapeDtypeStruct((N, D), jnp.float32),
        jax.ShapeDtypeStruct((N, 1), jnp.float32),
    ],
)


def _bn_relu(z, g, b):
    r = jnp.maximum(z, 0.0)
    m = jnp.mean(r, axis=0, keepdims=True)
    v = jnp.mean((r - m) ** 2, axis=0, keepdims=True)
    return (r - m) / jnp.sqrt(v + 1e-5) * g[None, :] + b[None, :]


def _tc2_body(agg_ref, h1_ref, dis_ref, b1_ref, g1_ref, be1_ref, w2_ref,
              h2_ref, hp2_ref):
    dis = dis_ref[...]
    z = dis * (agg_ref[0] + agg_ref[1]) + (dis * dis) * h1_ref[...] \
        + b1_ref[...][None, :]
    bn = _bn_relu(z, g1_ref[...], be1_ref[...])
    h2 = lax.dot_general(bn, w2_ref[...],
                         (((1,), (1,)), ((), ())),
                         preferred_element_type=jnp.float32)
    h2_ref[...] = h2
    hp2_ref[...] = h2 * dis


_tc2 = pl.pallas_call(
    _tc2_body,
    out_shape=[
        jax.ShapeDtypeStruct((N, D), jnp.float32),
        jax.ShapeDtypeStruct((N, D), jnp.float32),
    ],
)


def _tc3_body(agg_ref, h2_ref, dis_ref, b2_ref, g2_ref, be2_ref,
              wc_ref, bc_ref, out_ref):
    dis = dis_ref[...]
    z = dis * (agg_ref[0] + agg_ref[1]) + (dis * dis) * h2_ref[...] \
        + b2_ref[...][None, :]
    bn = _bn_relu(z, g2_ref[...], be2_ref[...])
    out_ref[...] = lax.dot_general(bn, wc_ref[...],
                                   (((1,), (1,)), ((), ())),
                                   preferred_element_type=jnp.float32) \
        + bc_ref[...][None, :]


_tc3 = pl.pallas_call(
    _tc3_body,
    out_shape=jax.ShapeDtypeStruct((N, D), jnp.float32),
)


def kernel(x, edge_index, edge_weight_param, W1, b1, g1, be1,
           W2, b2, g2, be2, Wc, bc):
    pad = EPAD - E
    zpad = jnp.zeros((pad,), jnp.int32)
    src3 = jnp.concatenate([edge_index[0], zpad]).reshape(NW, NCA, KA)
    dst3 = jnp.concatenate([edge_index[1], zpad]).reshape(NW, NCA, KA)
    idx4 = jnp.stack([src3, dst3], axis=2)
    ewp3 = jnp.concatenate(
        [edge_weight_param[:E],
         jnp.full((pad,), -100.0, jnp.float32)]).reshape(NW, NCA, KA)
    deg2 = _deg_kernel(edge_index[1], edge_weight_param[:E])
    h1, hp1, dis = _tc1(x, W1, deg2)
    agg1 = _agg_kernel(hp1, idx4, ewp3)
    h2, hp2 = _tc2(agg1, h1, dis, b1, g1, be1, W2)
    agg2 = _agg_kernel(hp2, idx4, ewp3)
    return _tc3(agg2, h2, dis, b2, g2, be2, Wc, bc)


# speedup vs baseline: 1.2891x; 1.2891x over previous
import functools

import jax
import jax.numpy as jnp
from jax import lax
from jax.experimental import pallas as pl
from jax.experimental.pallas import tpu as pltpu
from jax.experimental.pallas import tpu_sc as plsc

N = 10000
E = 320000
D = 128

NC = 2
NS = 16
NW = NC * NS
L = 16

EPW = E // NW
K = 80
NCHUNK = EPW // K
SPS = 624
TAIL = N - SPS * NS
ZROWS = 16
NZ = SPS // ZROWS


def _sc_mesh():
    return plsc.VectorSubcoreMesh(core_axis_name="c", subcore_axis_name="s")


@functools.partial(
    pl.kernel,
    mesh=_sc_mesh(),
    out_type=jax.ShapeDtypeStruct((NC, N, L), jnp.float32),
    scratch_types=[
        pltpu.VMEM((1, K), jnp.int32),
        pltpu.VMEM((K,), jnp.float32),
        pltpu.VMEM((K, L), jnp.float32),
        pltpu.VMEM((ZROWS, L), jnp.float32),
        pltpu.VMEM_SHARED((N, L), jnp.float32),
    ],
)
def _deg_kernel(dst_hbm, ewp_hbm, out_hbm, dstv, pbuf, msg, zbuf, deg_sp):
    c = lax.axis_index("c")
    s = lax.axis_index("s")
    w = s * NC + c
    zero = jnp.zeros((L,), jnp.float32)
    for r in range(ZROWS):
        zbuf[r, :] = zero

    def zloop(i, carry):
        pltpu.sync_copy(zbuf, deg_sp.at[pl.ds(s * SPS + i * ZROWS, ZROWS)])
        return carry

    lax.fori_loop(0, NZ, zloop, 0)

    @pl.when(s == NS - 1)
    def _():
        pltpu.sync_copy(zbuf, deg_sp.at[pl.ds(SPS * NS, TAIL)])

    plsc.subcore_barrier()

    def body(cidx, carry):
        base = w * EPW + cidx * K
        pltpu.sync_copy(dst_hbm.at[pl.ds(base, K)], dstv.at[0])
        pltpu.sync_copy(ewp_hbm.at[pl.ds(base, K)], pbuf)
        ramp = lax.iota(jnp.int32, L)
        onehot = [jnp.where(ramp == i, 1.0, 0.0).astype(jnp.float32)
                  for i in range(L)]

        def fill(g, carry2):
            wvec = jnp.exp(pbuf[pl.ds(g * L, L)])
            for i in range(L):
                msg[g * L + i, pl.ds(0, L)] = wvec * onehot[i]
            return carry2

        lax.fori_loop(0, K // L, fill, 0)
        pltpu.sync_copy(msg, deg_sp.at[dstv.at[0]], add=True)
        return carry

    lax.fori_loop(0, NCHUNK, body, 0)
    plsc.subcore_barrier()
    pltpu.sync_copy(deg_sp.at[pl.ds(s * SPS, SPS)],
                    out_hbm.at[c, pl.ds(s * SPS, SPS)])

    @pl.when(s == NS - 1)
    def _():
        pltpu.sync_copy(deg_sp.at[pl.ds(SPS * NS, TAIL)],
                        out_hbm.at[c, pl.ds(SPS * NS, TAIL)])


@functools.partial(
    pl.kernel,
    mesh=_sc_mesh(),
    out_type=jax.ShapeDtypeStruct((NC, N, D), jnp.float32),
    scratch_types=[
        pltpu.VMEM((2, K), jnp.int32),
        pltpu.VMEM((2, K), jnp.int32),
        pltpu.VMEM((K,), jnp.float32),
        pltpu.VMEM((K,), jnp.float32),
        pltpu.VMEM((K, D), jnp.float32),
        pltpu.VMEM((K, D), jnp.float32),
        pltpu.VMEM((ZROWS, D), jnp.float32),
        pltpu.VMEM_SHARED((N, D), jnp.float32),
        pltpu.SemaphoreType.DMA((2,)),
        pltpu.SemaphoreType.DMA((2,)),
    ],
)
def _agg_kernel(hp_hbm, idx_hbm, ewp_hbm, out_hbm,
                idxb0, idxb1, pbuf0, pbuf1, rows0, rows1, zbuf, agg_sp,
                gsems, isems):
    c = lax.axis_index("c")
    s = lax.axis_index("s")
    w = s * NC + c

    pltpu.sync_copy(idx_hbm.at[w, 0], idxb0)
    pltpu.sync_copy(ewp_hbm.at[w, 0], pbuf0)
    pltpu.async_copy(hp_hbm.at[idxb0.at[0]], rows0, gsems.at[0])

    zero = jnp.zeros((L,), jnp.float32)
    for r in range(ZROWS):
        for j in range(D // L):
            zbuf[r, pl.ds(j * L, L)] = zero

    nz = NZ + jnp.where(s == NS - 1, 1, 0)

    def zloop(i, carry):
        pltpu.sync_copy(zbuf, agg_sp.at[pl.ds(s * SPS + i * ZROWS, ZROWS)])
        return carry

    lax.fori_loop(0, nz, zloop, 0)
    plsc.subcore_barrier()

    def scale(buf, pbuf):
        def sgroup(g, carry2):
            wvec = jnp.exp(pbuf[pl.ds(g * L, L)])
            for i in range(L):
                wv = jnp.zeros((L,), jnp.float32) + wvec[i]
                e = g * L + i
                for j in range(D // L):
                    buf[e, pl.ds(j * L, L)] = buf[e, pl.ds(j * L, L)] * wv
            return carry2

        lax.fori_loop(0, K // L, sgroup, 0)

    def halfstep(t, idxb, pbuf, rows, idxn, pbufn, rowsn, bsel, bnext):
        @pl.when(t <= NCHUNK - 2)
        def _():
            pltpu.async_copy(idx_hbm.at[w, t + 1], idxn, isems.at[bnext])
            pltpu.async_copy(ewp_hbm.at[w, t + 1], pbufn, isems.at[bnext])

        pltpu.make_async_copy(hp_hbm.at[idxb.at[0]], rows,
                              gsems.at[bsel]).wait()
        scale(rows, pbuf)
        pltpu.async_copy(rows, agg_sp.at[idxb.at[1]], gsems.at[bsel],
                         add=True).wait()

        @pl.when(t <= NCHUNK - 2)
        def _():
            pltpu.make_async_copy(idx_hbm.at[w, 0], idxn,
                                  isems.at[bnext]).wait()
            pltpu.make_async_copy(ewp_hbm.at[w, 0], pbufn,
                                  isems.at[bnext]).wait()
            pltpu.async_copy(hp_hbm.at[idxn.at[0]], rowsn, gsems.at[bnext])

    def body(t, carry):
        @pl.when(t % 2 == 0)
        def _():
            halfstep(t, idxb0, pbuf0, rows0, idxb1, pbuf1, rows1, 0, 1)

        @pl.when(t % 2 == 1)
        def _():
            halfstep(t, idxb1, pbuf1, rows1, idxb0, pbuf0, rows0, 1, 0)

        return carry

    lax.fori_loop(0, NCHUNK, body, 0)
    plsc.subcore_barrier()

    ncp = NZ + jnp.where(s == NS - 1, 1, 0)

    def cloop(i, carry):
        pltpu.sync_copy(agg_sp.at[pl.ds(s * SPS + i * ZROWS, ZROWS)],
                        out_hbm.at[c, pl.ds(s * SPS + i * ZROWS, ZROWS)])
        return carry

    lax.fori_loop(0, ncp, cloop, 0)


def _tc1_body(x_ref, w1_ref, deg_ref, h1_ref, hp1_ref, dis_ref):
    h1 = lax.dot_general(x_ref[...], w1_ref[...],
                         (((1,), (1,)), ((), ())),
                         preferred_element_type=jnp.float32)
    deg = jnp.sum(deg_ref[0] + deg_ref[1], axis=-1, keepdims=True) + 1.0
    dis = lax.rsqrt(deg)
    h1_ref[...] = h1
    hp1_ref[...] = h1 * dis
    dis_ref[...] = dis


_tc1 = pl.pallas_call(
    _tc1_body,
    out_shape=[
        jax.ShapeDtypeStruct((N, D), jnp.float32),
        jax.ShapeDtypeStruct((N, D), jnp.float32),
        jax.ShapeDtypeStruct((N, 1), jnp.float32),
    ],
)


def _bn_relu(z, g, b):
    r = jnp.maximum(z, 0.0)
    m = jnp.mean(r, axis=0, keepdims=True)
    v = jnp.mean((r - m) ** 2, axis=0, keepdims=True)
    return (r - m) / jnp.sqrt(v + 1e-5) * g[None, :] + b[None, :]


def _tc2_body(agg_ref, h1_ref, dis_ref, b1_ref, g1_ref, be1_ref, w2_ref,
              h2_ref, hp2_ref):
    dis = dis_ref[...]
    z = dis * (agg_ref[0] + agg_ref[1]) + (dis * dis) * h1_ref[...] \
        + b1_ref[...][None, :]
    bn = _bn_relu(z, g1_ref[...], be1_ref[...])
    h2 = lax.dot_general(bn, w2_ref[...],
                         (((1,), (1,)), ((), ())),
                         preferred_element_type=jnp.float32)
    h2_ref[...] = h2
    hp2_ref[...] = h2 * dis


_tc2 = pl.pallas_call(
    _tc2_body,
    out_shape=[
        jax.ShapeDtypeStruct((N, D), jnp.float32),
        jax.ShapeDtypeStruct((N, D), jnp.float32),
    ],
)


def _tc3_body(agg_ref, h2_ref, dis_ref, b2_ref, g2_ref, be2_ref,
              wc_ref, bc_ref, out_ref):
    dis = dis_ref[...]
    z = dis * (agg_ref[0] + agg_ref[1]) + (dis * dis) * h2_ref[...] \
        + b2_ref[...][None, :]
    bn = _bn_relu(z, g2_ref[...], be2_ref[...])
    out_ref[...] = lax.dot_general(bn, wc_ref[...],
                                   (((1,), (1,)), ((), ())),
                                   preferred_element_type=jnp.float32) \
        + bc_ref[...][None, :]


_tc3 = pl.pallas_call(
    _tc3_body,
    out_shape=jax.ShapeDtypeStruct((N, D), jnp.float32),
)


def kernel(x, edge_index, edge_weight_param, W1, b1, g1, be1,
           W2, b2, g2, be2, Wc, bc):
    src3 = edge_index[0].reshape(NW, NCHUNK, K)
    dst3 = edge_index[1].reshape(NW, NCHUNK, K)
    idx4 = jnp.stack([src3, dst3], axis=2)
    ewp3 = edge_weight_param[:E].reshape(NW, NCHUNK, K)
    deg2 = _deg_kernel(edge_index[1], edge_weight_param[:E])
    h1, hp1, dis = _tc1(x, W1, deg2)
    agg1 = _agg_kernel(hp1, idx4, ewp3)
    h2, hp2 = _tc2(agg1, h1, dis, b1, g1, be1, W2)
    agg2 = _agg_kernel(hp2, idx4, ewp3)
    return _tc3(agg2, h2, dis, b2, g2, be2, Wc, bc)


# speedup vs baseline: 1.4833x; 1.1506x over previous
import functools

import jax
import jax.numpy as jnp
from jax import lax
from jax.experimental import pallas as pl
from jax.experimental.pallas import tpu as pltpu
from jax.experimental.pallas import tpu_sc as plsc

N = 10000
E = 320000
D = 128

NC = 2
NS = 16
NW = NC * NS
L = 16

EPW = E // NW
K = 80
NCHUNK = EPW // K
SPS = 624
TAIL = N - SPS * NS
ZROWS = 16
NZ = SPS // ZROWS


def _sc_mesh():
    return plsc.VectorSubcoreMesh(core_axis_name="c", subcore_axis_name="s")


@functools.partial(
    pl.kernel,
    mesh=_sc_mesh(),
    out_type=jax.ShapeDtypeStruct((NC, N, L), jnp.float32),
    scratch_types=[
        pltpu.VMEM((1, K), jnp.int32),
        pltpu.VMEM((K,), jnp.float32),
        pltpu.VMEM((K, L), jnp.float32),
        pltpu.VMEM((ZROWS, L), jnp.float32),
        pltpu.VMEM_SHARED((N, L), jnp.float32),
    ],
)
def _deg_kernel(dst_hbm, ewp_hbm, out_hbm, dstv, pbuf, msg, zbuf, deg_sp):
    c = lax.axis_index("c")
    s = lax.axis_index("s")
    w = s * NC + c
    zero = jnp.zeros((L,), jnp.float32)
    for r in range(ZROWS):
        zbuf[r, :] = zero

    def zloop(i, carry):
        pltpu.sync_copy(zbuf, deg_sp.at[pl.ds(s * SPS + i * ZROWS, ZROWS)])
        return carry

    lax.fori_loop(0, NZ, zloop, 0)

    @pl.when(s == NS - 1)
    def _():
        pltpu.sync_copy(zbuf, deg_sp.at[pl.ds(SPS * NS, TAIL)])

    plsc.subcore_barrier()

    def body(cidx, carry):
        base = w * EPW + cidx * K
        pltpu.sync_copy(dst_hbm.at[pl.ds(base, K)], dstv.at[0])
        pltpu.sync_copy(ewp_hbm.at[pl.ds(base, K)], pbuf)
        ramp = lax.iota(jnp.int32, L)
        onehot = [jnp.where(ramp == i, 1.0, 0.0).astype(jnp.float32)
                  for i in range(L)]

        def fill(g, carry2):
            wvec = jnp.exp(pbuf[pl.ds(g * L, L)])
            for i in range(L):
                msg[g * L + i, pl.ds(0, L)] = wvec * onehot[i]
            return carry2

        lax.fori_loop(0, K // L, fill, 0)
        pltpu.sync_copy(msg, deg_sp.at[dstv.at[0]], add=True)
        return carry

    lax.fori_loop(0, NCHUNK, body, 0)
    plsc.subcore_barrier()
    pltpu.sync_copy(deg_sp.at[pl.ds(s * SPS, SPS)],
                    out_hbm.at[c, pl.ds(s * SPS, SPS)])

    @pl.when(s == NS - 1)
    def _():
        pltpu.sync_copy(deg_sp.at[pl.ds(SPS * NS, TAIL)],
                        out_hbm.at[c, pl.ds(SPS * NS, TAIL)])


@functools.partial(
    pl.kernel,
    mesh=_sc_mesh(),
    out_type=jax.ShapeDtypeStruct((NC, N, D), jnp.float32),
    scratch_types=[
        pltpu.VMEM((2, K), jnp.int32),
        pltpu.VMEM((2, K), jnp.int32),
        pltpu.VMEM((K,), jnp.float32),
        pltpu.VMEM((K,), jnp.float32),
        pltpu.VMEM((K, D), jnp.float32),
        pltpu.VMEM((K, D), jnp.float32),
        pltpu.VMEM((ZROWS, D), jnp.float32),
        pltpu.VMEM_SHARED((N, D), jnp.float32),
        pltpu.VMEM((1, K), jnp.int32),
        pltpu.VMEM((1, K), jnp.int32),
        pltpu.SemaphoreType.DMA((2,)),
        pltpu.SemaphoreType.DMA((2,)),
        pltpu.SemaphoreType.DMA((2,)),
    ],
)
def _agg_kernel(hp_hbm, idx_hbm, ewp_hbm, out_hbm,
                idxb0, idxb1, pbuf0, pbuf1, rows0, rows1, zbuf, agg_sp,
                sidx0, sidx1, gsems, isems, ssems):
    c = lax.axis_index("c")
    s = lax.axis_index("s")
    w = s * NC + c

    pltpu.sync_copy(idx_hbm.at[w, 0], idxb0)
    pltpu.sync_copy(ewp_hbm.at[w, 0], pbuf0)
    pltpu.async_copy(hp_hbm.at[idxb0.at[0]], rows0, gsems.at[0])

    zero = jnp.zeros((L,), jnp.float32)
    for r in range(ZROWS):
        for j in range(D // L):
            zbuf[r, pl.ds(j * L, L)] = zero

    nz = NZ + jnp.where(s == NS - 1, 1, 0)

    def zloop(i, carry):
        pltpu.sync_copy(zbuf, agg_sp.at[pl.ds(s * SPS + i * ZROWS, ZROWS)])
        return carry

    lax.fori_loop(0, nz, zloop, 0)
    plsc.subcore_barrier()

    def scale(buf, pbuf):
        def sgroup(g, carry2):
            wvec = jnp.exp(pbuf[pl.ds(g * L, L)])
            for i in range(L):
                wv = jnp.zeros((L,), jnp.float32) + wvec[i]
                e = g * L + i
                for j in range(D // L):
                    buf[e, pl.ds(j * L, L)] = buf[e, pl.ds(j * L, L)] * wv
            return carry2

        lax.fori_loop(0, K // L, sgroup, 0)

    def halfstep(t, idxb, pbuf, rows, sidx, idxn, pbufn, rowsn, sidxn,
                 bsel, bnext):
        @pl.when(t <= NCHUNK - 2)
        def _():
            pltpu.async_copy(idx_hbm.at[w, t + 1], idxn, isems.at[bnext])
            pltpu.async_copy(ewp_hbm.at[w, t + 1], pbufn, isems.at[bnext])

        pltpu.make_async_copy(hp_hbm.at[idxb.at[0]], rows,
                              gsems.at[bsel]).wait()
        scale(rows, pbuf)

        @pl.when(t >= 1)
        def _():
            pltpu.make_async_copy(rowsn, agg_sp.at[sidxn.at[0]],
                                  ssems.at[bnext]).wait()

        for g in range(K // L):
            sidx[0, pl.ds(g * L, L)] = idxb[1, pl.ds(g * L, L)]
        pltpu.async_copy(rows, agg_sp.at[sidx.at[0]], ssems.at[bsel],
                         add=True)

        @pl.when(t <= NCHUNK - 2)
        def _():
            pltpu.make_async_copy(idx_hbm.at[w, 0], idxn,
                                  isems.at[bnext]).wait()
            pltpu.make_async_copy(ewp_hbm.at[w, 0], pbufn,
                                  isems.at[bnext]).wait()
            pltpu.async_copy(hp_hbm.at[idxn.at[0]], rowsn, gsems.at[bnext])

    def body(t, carry):
        @pl.when(t % 2 == 0)
        def _():
            halfstep(t, idxb0, pbuf0, rows0, sidx0, idxb1, pbuf1, rows1,
                     sidx1, 0, 1)

        @pl.when(t % 2 == 1)
        def _():
            halfstep(t, idxb1, pbuf1, rows1, sidx1, idxb0, pbuf0, rows0,
                     sidx0, 1, 0)

        return carry

    lax.fori_loop(0, NCHUNK, body, 0)
    pltpu.make_async_copy(rows0, agg_sp.at[sidx0.at[0]], ssems.at[0]).wait()
    plsc.subcore_barrier()

    ncp = NZ + jnp.where(s == NS - 1, 1, 0)

    def cloop(i, carry):
        pltpu.sync_copy(agg_sp.at[pl.ds(s * SPS + i * ZROWS, ZROWS)],
                        out_hbm.at[c, pl.ds(s * SPS + i * ZROWS, ZROWS)])
        return carry

    lax.fori_loop(0, ncp, cloop, 0)


def _tc1_body(x_ref, w1_ref, deg_ref, h1_ref, hp1_ref, dis_ref):
    h1 = lax.dot_general(x_ref[...], w1_ref[...],
                         (((1,), (1,)), ((), ())),
                         preferred_element_type=jnp.float32)
    deg = jnp.sum(deg_ref[0] + deg_ref[1], axis=-1, keepdims=True) + 1.0
    dis = lax.rsqrt(deg)
    h1_ref[...] = h1
    hp1_ref[...] = h1 * dis
    dis_ref[...] = dis


_tc1 = pl.pallas_call(
    _tc1_body,
    out_shape=[
        jax.ShapeDtypeStruct((N, D), jnp.float32),
        jax.ShapeDtypeStruct((N, D), jnp.float32),
        jax.ShapeDtypeStruct((N, 1), jnp.float32),
    ],
)


def _bn_relu(z, g, b):
    r = jnp.maximum(z, 0.0)
    m = jnp.mean(r, axis=0, keepdims=True)
    v = jnp.mean((r - m) ** 2, axis=0, keepdims=True)
    return (r - m) / jnp.sqrt(v + 1e-5) * g[None, :] + b[None, :]


def _tc2_body(agg_ref, h1_ref, dis_ref, b1_ref, g1_ref, be1_ref, w2_ref,
              h2_ref, hp2_ref):
    dis = dis_ref[...]
    z = dis * (agg_ref[0] + agg_ref[1]) + (dis * dis) * h1_ref[...] \
        + b1_ref[...][None, :]
    bn = _bn_relu(z, g1_ref[...], be1_ref[...])
    h2 = lax.dot_general(bn, w2_ref[...],
                         (((1,), (1,)), ((), ())),
                         preferred_element_type=jnp.float32)
    h2_ref[...] = h2
    hp2_ref[...] = h2 * dis


_tc2 = pl.pallas_call(
    _tc2_body,
    out_shape=[
        jax.ShapeDtypeStruct((N, D), jnp.float32),
        jax.ShapeDtypeStruct((N, D), jnp.float32),
    ],
)


def _tc3_body(agg_ref, h2_ref, dis_ref, b2_ref, g2_ref, be2_ref,
              wc_ref, bc_ref, out_ref):
    dis = dis_ref[...]
    z = dis * (agg_ref[0] + agg_ref[1]) + (dis * dis) * h2_ref[...] \
        + b2_ref[...][None, :]
    bn = _bn_relu(z, g2_ref[...], be2_ref[...])
    out_ref[...] = lax.dot_general(bn, wc_ref[...],
                                   (((1,), (1,)), ((), ())),
                                   preferred_element_type=jnp.float32) \
        + bc_ref[...][None, :]


_tc3 = pl.pallas_call(
    _tc3_body,
    out_shape=jax.ShapeDtypeStruct((N, D), jnp.float32),
)


def kernel(x, edge_index, edge_weight_param, W1, b1, g1, be1,
           W2, b2, g2, be2, Wc, bc):
    src3 = edge_index[0].reshape(NW, NCHUNK, K)
    dst3 = edge_index[1].reshape(NW, NCHUNK, K)
    idx4 = jnp.stack([src3, dst3], axis=2)
    ewp3 = edge_weight_param[:E].reshape(NW, NCHUNK, K)
    deg2 = _deg_kernel(edge_index[1], edge_weight_param[:E])
    h1, hp1, dis = _tc1(x, W1, deg2)
    agg1 = _agg_kernel(hp1, idx4, ewp3)
    h2, hp2 = _tc2(agg1, h1, dis, b1, g1, be1, W2)
    agg2 = _agg_kernel(hp2, idx4, ewp3)
    return _tc3(agg2, h2, dis, b2, g2, be2, Wc, bc)
